# Initial kernel scaffold; baseline (speedup 1.0000x reference)
import jax
import jax.numpy as jnp
from jax.experimental import pallas as pl


def kernel(input_nodes, in_degree, out_degree, atom_w, in_w, out_w, graph_token_w):
    raise NotImplementedError("write your pallas kernel here")


# speedup vs baseline: 1.7537x; 1.7537x over previous
import functools

import jax
import jax.numpy as jnp
from jax import lax
from jax.experimental import pallas as pl
from jax.experimental.pallas import tpu as pltpu
from jax.experimental.pallas import tpu_sc as plsc

B, N, F, H = 256, 128, 9, 768

_info = plsc.get_sparse_core_info()
NC, NS = _info.num_cores, _info.num_subcores
NW = NC * NS
TOKENS = B * N
TPW = TOKENS // NW
T = 8
CHUNKS = TPW // T
GPW = B // NW
LANES = 16
JCH = H // LANES


def _sc_kernel(nodes_hbm, ind_hbm, outd_hbm, atom_hbm, inw_hbm, outw_hbm,
               gtok_hbm, out_hbm,
               idx_v, din_v, dout_v, rows_v, inr_v, outr_v, acc_v, gt_v,
               sem, sem2, sem3):
    wid = lax.axis_index("s") * NC + lax.axis_index("c")
    tok0 = wid * TPW

    pltpu.sync_copy(nodes_hbm.at[pl.ds(tok0 * F, TPW * F)], idx_v)
    pltpu.sync_copy(ind_hbm.at[pl.ds(tok0, TPW)], din_v)
    pltpu.sync_copy(outd_hbm.at[pl.ds(tok0, TPW)], dout_v)
    pltpu.sync_copy(gtok_hbm, gt_v)

    g0 = wid * GPW
    for gi in range(GPW):
        pltpu.sync_copy(gt_v, out_hbm.at[pl.ds((g0 + gi) * (N + 1) * H, H)])

    def chunk_body(c, _):
        cp1 = pltpu.async_copy(atom_hbm.at[idx_v.at[pl.ds(c * T * F, T * F)]],
                               rows_v, sem)
        cp2 = pltpu.async_copy(inw_hbm.at[din_v.at[pl.ds(c * T, T)]],
                               inr_v, sem2)
        cp3 = pltpu.async_copy(outw_hbm.at[dout_v.at[pl.ds(c * T, T)]],
                               outr_v, sem3)
        cp1.wait()
        cp2.wait()
        cp3.wait()

        def tok_body(t, _):
            r0 = t * F

            def lane_body(j, _):
                s = pl.ds(j * LANES, LANES)
                acc = rows_v[r0, s]
                for f in range(1, F):
                    acc = acc + rows_v[r0 + f, s]
                acc = acc + inr_v[t, s] + outr_v[t, s]
                acc_v[pl.ds(t * H + j * LANES, LANES)] = acc
                return 0

            lax.fori_loop(0, JCH, lane_body, 0, unroll=4)
            return 0

        lax.fori_loop(0, T, tok_body, 0)

        tk = tok0 + c * T
        g = tk // N
        n0 = tk - g * N
        pltpu.sync_copy(acc_v,
                        out_hbm.at[pl.ds((g * (N + 1) + n0 + 1) * H, T * H)])
        return 0

    lax.fori_loop(0, CHUNKS, chunk_body, 0)


@jax.jit
def _run(nodes_flat, ind_flat, outd_flat, atom_w, in_w, out_w, graph_token_w):
    mesh = plsc.VectorSubcoreMesh(core_axis_name="c", subcore_axis_name="s")
    kfn = functools.partial(
        pl.kernel,
        mesh=mesh,
        out_type=jax.ShapeDtypeStruct((B * (N + 1) * H,), jnp.float32),
        scratch_types=[
            pltpu.VMEM((TPW * F,), jnp.int32),
            pltpu.VMEM((TPW,), jnp.int32),
            pltpu.VMEM((TPW,), jnp.int32),
            pltpu.VMEM((T * F, H), jnp.float32),
            pltpu.VMEM((T, H), jnp.float32),
            pltpu.VMEM((T, H), jnp.float32),
            pltpu.VMEM((T * H,), jnp.float32),
            pltpu.VMEM((H,), jnp.float32),
            pltpu.SemaphoreType.DMA,
            pltpu.SemaphoreType.DMA,
            pltpu.SemaphoreType.DMA,
        ],
    )(_sc_kernel)
    flat = kfn(nodes_flat, ind_flat, outd_flat, atom_w, in_w, out_w,
               graph_token_w)
    return flat.reshape(B, N + 1, H)


def kernel(input_nodes, in_degree, out_degree, atom_w, in_w, out_w,
           graph_token_w):
    nodes_flat = jnp.asarray(input_nodes, jnp.int32).reshape(-1)
    ind_flat = jnp.asarray(in_degree, jnp.int32).reshape(-1)
    outd_flat = jnp.asarray(out_degree, jnp.int32).reshape(-1)
    return _run(nodes_flat, ind_flat, outd_flat, atom_w, in_w, out_w,
                graph_token_w.reshape(-1))


# speedup vs baseline: 2.0816x; 1.1870x over previous
import functools

import jax
import jax.numpy as jnp
from jax import lax
from jax.experimental import pallas as pl
from jax.experimental.pallas import tpu as pltpu
from jax.experimental.pallas import tpu_sc as plsc

B, N, F, H = 256, 128, 9, 768

_info = plsc.get_sparse_core_info()
NC, NS = _info.num_cores, _info.num_subcores
NW = NC * NS
TOKENS = B * N
TPW = TOKENS // NW
T = 8
CHUNKS = TPW // T
GCHUNKS = TOKENS // T
CW = T * F + 2 * T
GPW = B // NW
LANES = 16
JCH = H // LANES
UNROLL = 4


def _sc_kernel(comb_hbm, atom_hbm, inw_hbm, outw_hbm, gtok_hbm, out_hbm,
               idx_a, idx_b, rows_a, rows_b, deg_v, acc_v, gt_v,
               sem_a, sem_b, sem_d, sem_o, sem_ia, sem_ib):
    wid = lax.axis_index("s") * NC + lax.axis_index("c")
    gc0 = wid * CHUNKS

    def stage_idx(c, idx_ref, sem):
        pltpu.async_copy(comb_hbm.at[pl.ds((gc0 + c) * CW, CW)], idx_ref,
                         sem)

    def wait_idx(idx_ref, sem):
        pltpu.make_async_copy(comb_hbm.at[pl.ds(0, CW)], idx_ref,
                              sem).wait()

    def fire_atoms(idx_ref, rows_ref, sem):
        pltpu.async_copy(atom_hbm.at[idx_ref.at[pl.ds(0, T * F)]],
                         rows_ref, sem)

    def wait_atoms(rows_ref, sem):
        pltpu.make_async_copy(atom_hbm.at[pl.ds(0, T * F)], rows_ref,
                              sem).wait()

    def fire_deg(idx_ref):
        pltpu.async_copy(inw_hbm.at[idx_ref.at[pl.ds(T * F, T)]],
                         deg_v.at[pl.ds(0, T)], sem_d)
        pltpu.async_copy(outw_hbm.at[idx_ref.at[pl.ds(T * F + T, T)]],
                         deg_v.at[pl.ds(T, T)], sem_d)

    def wait_deg():
        pltpu.make_async_copy(inw_hbm.at[pl.ds(0, T)],
                              deg_v.at[pl.ds(0, T)], sem_d).wait()
        pltpu.make_async_copy(outw_hbm.at[pl.ds(0, T)],
                              deg_v.at[pl.ds(T, T)], sem_d).wait()

    def wait_out():
        pltpu.make_async_copy(out_hbm.at[pl.ds(0, T * H)], acc_v,
                              sem_o).wait()

    def compute(rows_ref):
        def tok_body(t, _):
            r0 = t * F

            def lane_body(j, _):
                s = pl.ds(j * LANES, LANES)
                acc = rows_ref[r0, s]
                for f in range(1, F):
                    acc = acc + rows_ref[r0 + f, s]
                acc = acc + deg_v[t, s] + deg_v[t + T, s]
                acc_v[pl.ds(t * H + j * LANES, LANES)] = acc
                return 0

            lax.fori_loop(0, JCH, lane_body, 0, unroll=UNROLL)
            return 0

        lax.fori_loop(0, T, tok_body, 0)

    def store_out(c):
        g = wid * GPW + lax.shift_right_logical(c, 4)
        n0 = lax.shift_left(lax.bitwise_and(c, 15), 3)
        off = (g * (N + 1) + n0 + 1) * H
        pltpu.async_copy(acc_v, out_hbm.at[pl.ds(off, T * H)], sem_o)

    stage_idx(0, idx_a, sem_ia)
    wait_idx(idx_a, sem_ia)
    fire_atoms(idx_a, rows_a, sem_a)
    fire_deg(idx_a)
    stage_idx(1, idx_b, sem_ib)

    pltpu.sync_copy(gtok_hbm, gt_v)
    g0 = wid * GPW
    for gi in range(GPW):
        pltpu.sync_copy(gt_v, out_hbm.at[pl.ds((g0 + gi) * (N + 1) * H, H)])

    def pair_body(i, _):
        c0 = i * 2
        wait_idx(idx_b, sem_ib)
        fire_atoms(idx_b, rows_b, sem_b)
        wait_atoms(rows_a, sem_a)
        wait_deg()

        @pl.when(i > 0)
        def _():
            wait_out()

        stage_idx(c0 + 2, idx_a, sem_ia)
        compute(rows_a)
        fire_deg(idx_b)
        store_out(c0)
        wait_idx(idx_a, sem_ia)
        fire_atoms(idx_a, rows_a, sem_a)
        wait_atoms(rows_b, sem_b)
        wait_deg()
        wait_out()
        stage_idx(c0 + 3, idx_b, sem_ib)
        compute(rows_b)
        fire_deg(idx_a)
        store_out(c0 + 1)
        return 0

    lax.fori_loop(0, CHUNKS // 2, pair_body, 0)

    wait_idx(idx_b, sem_ib)
    wait_atoms(rows_a, sem_a)
    wait_deg()
    wait_out()


@jax.jit
def _run(comb, atom_w, in_w, out_w, gtok_flat):
    mesh = plsc.VectorSubcoreMesh(core_axis_name="c", subcore_axis_name="s")
    kfn = functools.partial(
        pl.kernel,
        mesh=mesh,
        out_type=jax.ShapeDtypeStruct((B * (N + 1) * H,), jnp.float32),
        scratch_types=[
            pltpu.VMEM((CW,), jnp.int32),
            pltpu.VMEM((CW,), jnp.int32),
            pltpu.VMEM((T * F, H), jnp.float32),
            pltpu.VMEM((T * F, H), jnp.float32),
            pltpu.VMEM((2 * T, H), jnp.float32),
            pltpu.VMEM((T * H,), jnp.float32),
            pltpu.VMEM((H,), jnp.float32),
            pltpu.SemaphoreType.DMA,
            pltpu.SemaphoreType.DMA,
            pltpu.SemaphoreType.DMA,
            pltpu.SemaphoreType.DMA,
            pltpu.SemaphoreType.DMA,
            pltpu.SemaphoreType.DMA,
        ],
    )(_sc_kernel)
    flat = kfn(comb, atom_w, in_w, out_w, gtok_flat)
    return flat.reshape(B, N + 1, H)


def kernel(input_nodes, in_degree, out_degree, atom_w, in_w, out_w,
           graph_token_w):
    atoms = jnp.asarray(input_nodes, jnp.int32).reshape(GCHUNKS, T * F)
    ins = jnp.asarray(in_degree, jnp.int32).reshape(GCHUNKS, T)
    outs = jnp.asarray(out_degree, jnp.int32).reshape(GCHUNKS, T)
    comb = jnp.concatenate([atoms, ins, outs], axis=1)
    comb = jnp.concatenate(
        [comb, jnp.zeros((2, CW), jnp.int32)], axis=0).reshape(-1)
    return _run(comb, atom_w, in_w, out_w, graph_token_w.reshape(-1))


# speedup vs baseline: 2.1211x; 1.0190x over previous
import functools

import jax
import jax.numpy as jnp
from jax import lax
from jax.experimental import pallas as pl
from jax.experimental.pallas import tpu as pltpu
from jax.experimental.pallas import tpu_sc as plsc

B, N, F, H = 256, 128, 9, 768

_info = plsc.get_sparse_core_info()
NC, NS = _info.num_cores, _info.num_subcores
NW = NC * NS
TOKENS = B * N
TPW = TOKENS // NW
T = 8
CHUNKS = TPW // T
GCHUNKS = TOKENS // T
CW = T * F + 2 * T
GPW = B // NW
LANES = 16
JCH = H // LANES
UNROLL = 4


def _sc_kernel(comb_hbm, atom_hbm, inw_hbm, outw_hbm, gtok_hbm, out_hbm,
               idx_a, idx_b, rows_a, rows_b, deg_v, acc_v, gt_v,
               sem_a, sem_b, sem_d, sem_o, sem_ia, sem_ib):
    wid = lax.axis_index("s") * NC + lax.axis_index("c")
    gc0 = wid * CHUNKS

    def stage_idx(c, idx_ref, sem):
        pltpu.async_copy(comb_hbm.at[pl.ds((gc0 + c) * CW, CW)], idx_ref,
                         sem)

    def wait_idx(idx_ref, sem):
        pltpu.make_async_copy(comb_hbm.at[pl.ds(0, CW)], idx_ref,
                              sem).wait()

    def fire_atoms(idx_ref, rows_ref, sem):
        pltpu.async_copy(atom_hbm.at[idx_ref.at[pl.ds(0, T * F)]],
                         rows_ref, sem)

    def wait_atoms(rows_ref, sem):
        pltpu.make_async_copy(atom_hbm.at[pl.ds(0, T * F)], rows_ref,
                              sem).wait()

    def fire_deg(idx_ref):
        pltpu.async_copy(inw_hbm.at[idx_ref.at[pl.ds(T * F, T)]],
                         deg_v.at[pl.ds(0, T)], sem_d)
        pltpu.async_copy(outw_hbm.at[idx_ref.at[pl.ds(T * F + T, T)]],
                         deg_v.at[pl.ds(T, T)], sem_d)

    def wait_deg():
        pltpu.make_async_copy(inw_hbm.at[pl.ds(0, T)],
                              deg_v.at[pl.ds(0, T)], sem_d).wait()
        pltpu.make_async_copy(outw_hbm.at[pl.ds(0, T)],
                              deg_v.at[pl.ds(T, T)], sem_d).wait()

    def wait_out():
        pltpu.make_async_copy(out_hbm.at[pl.ds(0, T * H)], acc_v,
                              sem_o).wait()

    def compute(rows_ref):
        def lane_body(j, _):
            s = pl.ds(j * LANES, LANES)
            for t in range(T):
                r0 = t * F
                acc = rows_ref[r0, s]
                for f in range(1, F):
                    acc = acc + rows_ref[r0 + f, s]
                acc = acc + deg_v[t, s] + deg_v[t + T, s]
                acc_v[pl.ds(t * H + j * LANES, LANES)] = acc
            return 0

        lax.fori_loop(0, JCH, lane_body, 0)

    def store_out(c):
        g = wid * GPW + lax.shift_right_logical(c, 4)
        n0 = lax.shift_left(lax.bitwise_and(c, 15), 3)
        off = (g * (N + 1) + n0 + 1) * H
        pltpu.async_copy(acc_v, out_hbm.at[pl.ds(off, T * H)], sem_o)

    stage_idx(0, idx_a, sem_ia)
    wait_idx(idx_a, sem_ia)
    fire_atoms(idx_a, rows_a, sem_a)
    fire_deg(idx_a)
    stage_idx(1, idx_b, sem_ib)

    pltpu.sync_copy(gtok_hbm, gt_v)
    g0 = wid * GPW
    for gi in range(GPW):
        pltpu.sync_copy(gt_v, out_hbm.at[pl.ds((g0 + gi) * (N + 1) * H, H)])

    def pair_body(i, _):
        c0 = i * 2
        wait_idx(idx_b, sem_ib)
        fire_atoms(idx_b, rows_b, sem_b)
        wait_atoms(rows_a, sem_a)
        wait_deg()

        @pl.when(i > 0)
        def _():
            wait_out()

        stage_idx(c0 + 2, idx_a, sem_ia)
        compute(rows_a)
        fire_deg(idx_b)
        store_out(c0)
        wait_idx(idx_a, sem_ia)
        fire_atoms(idx_a, rows_a, sem_a)
        wait_atoms(rows_b, sem_b)
        wait_deg()
        wait_out()
        stage_idx(c0 + 3, idx_b, sem_ib)
        compute(rows_b)
        fire_deg(idx_a)
        store_out(c0 + 1)
        return 0

    lax.fori_loop(0, CHUNKS // 2, pair_body, 0)

    wait_idx(idx_b, sem_ib)
    wait_atoms(rows_a, sem_a)
    wait_deg()
    wait_out()


@jax.jit
def _run(comb, atom_w, in_w, out_w, gtok_flat):
    mesh = plsc.VectorSubcoreMesh(core_axis_name="c", subcore_axis_name="s")
    kfn = functools.partial(
        pl.kernel,
        mesh=mesh,
        out_type=jax.ShapeDtypeStruct((B * (N + 1) * H,), jnp.float32),
        scratch_types=[
            pltpu.VMEM((CW,), jnp.int32),
            pltpu.VMEM((CW,), jnp.int32),
            pltpu.VMEM((T * F, H), jnp.float32),
            pltpu.VMEM((T * F, H), jnp.float32),
            pltpu.VMEM((2 * T, H), jnp.float32),
            pltpu.VMEM((T * H,), jnp.float32),
            pltpu.VMEM((H,), jnp.float32),
            pltpu.SemaphoreType.DMA,
            pltpu.SemaphoreType.DMA,
            pltpu.SemaphoreType.DMA,
            pltpu.SemaphoreType.DMA,
            pltpu.SemaphoreType.DMA,
            pltpu.SemaphoreType.DMA,
        ],
    )(_sc_kernel)
    flat = kfn(comb, atom_w, in_w, out_w, gtok_flat)
    return flat.reshape(B, N + 1, H)


def kernel(input_nodes, in_degree, out_degree, atom_w, in_w, out_w,
           graph_token_w):
    atoms = jnp.asarray(input_nodes, jnp.int32).reshape(GCHUNKS, T * F)
    ins = jnp.asarray(in_degree, jnp.int32).reshape(GCHUNKS, T)
    outs = jnp.asarray(out_degree, jnp.int32).reshape(GCHUNKS, T)
    comb = jnp.concatenate([atoms, ins, outs], axis=1)
    comb = jnp.concatenate(
        [comb, jnp.zeros((2, CW), jnp.int32)], axis=0).reshape(-1)
    return _run(comb, atom_w, in_w, out_w, graph_token_w.reshape(-1))


# speedup vs baseline: 3.1552x; 1.4875x over previous
import functools

import jax
import jax.numpy as jnp
from jax import lax
from jax.experimental import pallas as pl
from jax.experimental.pallas import tpu as pltpu
from jax.experimental.pallas import tpu_sc as plsc

B, N, F, H = 256, 128, 9, 768

_info = plsc.get_sparse_core_info()
NC, NS = _info.num_cores, _info.num_subcores
NW = NC * NS
TOKENS = B * N
TPW = TOKENS // NW
T = 8
CHUNKS = TPW // T
GCHUNKS = TOKENS // T
CW = T * F + 2 * T
GPW = B // NW
LANES = 16
JCH = H // LANES
UNROLL = 4


def _sc_kernel(comb_hbm, atom_hbm, inw_hbm, outw_hbm, gtok_hbm, out_hbm,
               idx_a, idx_b, rows_a, rows_b, deg_v, acc_v, gt_v,
               sem_a, sem_b, sem_d, sem_o, sem_ia, sem_ib):
    wid = lax.axis_index("s") * NC + lax.axis_index("c")
    gc0 = wid * CHUNKS

    def stage_idx(c, idx_ref, sem):
        pltpu.async_copy(comb_hbm.at[pl.ds((gc0 + c) * CW, CW)], idx_ref,
                         sem)

    def wait_idx(idx_ref, sem):
        pltpu.make_async_copy(comb_hbm.at[pl.ds(0, CW)], idx_ref,
                              sem).wait()

    def fire_atoms(idx_ref, rows_ref, sem):
        pltpu.async_copy(atom_hbm.at[idx_ref.at[pl.ds(0, T * F)]],
                         rows_ref, sem)

    def wait_atoms(rows_ref, sem):
        pltpu.make_async_copy(atom_hbm.at[pl.ds(0, T * F)], rows_ref,
                              sem).wait()

    def fire_deg(idx_ref):
        pltpu.async_copy(inw_hbm.at[idx_ref.at[pl.ds(T * F, T)]],
                         deg_v.at[pl.ds(0, T)], sem_d)
        pltpu.async_copy(outw_hbm.at[idx_ref.at[pl.ds(T * F + T, T)]],
                         deg_v.at[pl.ds(T, T)], sem_d)

    def wait_deg():
        pltpu.make_async_copy(inw_hbm.at[pl.ds(0, T)],
                              deg_v.at[pl.ds(0, T)], sem_d).wait()
        pltpu.make_async_copy(outw_hbm.at[pl.ds(0, T)],
                              deg_v.at[pl.ds(T, T)], sem_d).wait()

    def wait_out():
        pltpu.make_async_copy(out_hbm.at[pl.ds(0, T * H)], acc_v,
                              sem_o).wait()

    def compute(rows_ref):
        def lane_body(j, _):
            s = pl.ds(j * LANES, LANES)
            accs = [rows_ref[t * F, s] for t in range(T)]
            for f in range(1, F):
                for t in range(T):
                    accs[t] = accs[t] + rows_ref[t * F + f, s]
            for t in range(T):
                accs[t] = accs[t] + deg_v[t, s]
            for t in range(T):
                accs[t] = accs[t] + deg_v[t + T, s]
            for t in range(T):
                acc_v[pl.ds(t * H + j * LANES, LANES)] = accs[t]
            return 0

        lax.fori_loop(0, JCH, lane_body, 0)

    def store_out(c):
        g = wid * GPW + lax.shift_right_logical(c, 4)
        n0 = lax.shift_left(lax.bitwise_and(c, 15), 3)
        off = (g * (N + 1) + n0 + 1) * H
        pltpu.async_copy(acc_v, out_hbm.at[pl.ds(off, T * H)], sem_o)

    stage_idx(0, idx_a, sem_ia)
    wait_idx(idx_a, sem_ia)
    fire_atoms(idx_a, rows_a, sem_a)
    fire_deg(idx_a)
    stage_idx(1, idx_b, sem_ib)

    pltpu.sync_copy(gtok_hbm, gt_v)
    g0 = wid * GPW
    for gi in range(GPW):
        pltpu.sync_copy(gt_v, out_hbm.at[pl.ds((g0 + gi) * (N + 1) * H, H)])

    def pair_body(i, _):
        c0 = i * 2
        wait_idx(idx_b, sem_ib)
        fire_atoms(idx_b, rows_b, sem_b)
        wait_atoms(rows_a, sem_a)
        wait_deg()

        @pl.when(i > 0)
        def _():
            wait_out()

        stage_idx(c0 + 2, idx_a, sem_ia)
        compute(rows_a)
        fire_deg(idx_b)
        store_out(c0)
        wait_idx(idx_a, sem_ia)
        fire_atoms(idx_a, rows_a, sem_a)
        wait_atoms(rows_b, sem_b)
        wait_deg()
        wait_out()
        stage_idx(c0 + 3, idx_b, sem_ib)
        compute(rows_b)
        fire_deg(idx_a)
        store_out(c0 + 1)
        return 0

    lax.fori_loop(0, CHUNKS // 2, pair_body, 0)

    wait_idx(idx_b, sem_ib)
    wait_atoms(rows_a, sem_a)
    wait_deg()
    wait_out()


@jax.jit
def _run(comb, atom_w, in_w, out_w, gtok_flat):
    mesh = plsc.VectorSubcoreMesh(core_axis_name="c", subcore_axis_name="s")
    kfn = functools.partial(
        pl.kernel,
        mesh=mesh,
        out_type=jax.ShapeDtypeStruct((B * (N + 1) * H,), jnp.float32),
        scratch_types=[
            pltpu.VMEM((CW,), jnp.int32),
            pltpu.VMEM((CW,), jnp.int32),
            pltpu.VMEM((T * F, H), jnp.float32),
            pltpu.VMEM((T * F, H), jnp.float32),
            pltpu.VMEM((2 * T, H), jnp.float32),
            pltpu.VMEM((T * H,), jnp.float32),
            pltpu.VMEM((H,), jnp.float32),
            pltpu.SemaphoreType.DMA,
            pltpu.SemaphoreType.DMA,
            pltpu.SemaphoreType.DMA,
            pltpu.SemaphoreType.DMA,
            pltpu.SemaphoreType.DMA,
            pltpu.SemaphoreType.DMA,
        ],
    )(_sc_kernel)
    flat = kfn(comb, atom_w, in_w, out_w, gtok_flat)
    return flat.reshape(B, N + 1, H)


def kernel(input_nodes, in_degree, out_degree, atom_w, in_w, out_w,
           graph_token_w):
    atoms = jnp.asarray(input_nodes, jnp.int32).reshape(GCHUNKS, T * F)
    ins = jnp.asarray(in_degree, jnp.int32).reshape(GCHUNKS, T)
    outs = jnp.asarray(out_degree, jnp.int32).reshape(GCHUNKS, T)
    comb = jnp.concatenate([atoms, ins, outs], axis=1)
    comb = jnp.concatenate(
        [comb, jnp.zeros((2, CW), jnp.int32)], axis=0).reshape(-1)
    return _run(comb, atom_w, in_w, out_w, graph_token_w.reshape(-1))


# speedup vs baseline: 3.1940x; 1.0123x over previous
import functools

import jax
import jax.numpy as jnp
from jax import lax
from jax.experimental import pallas as pl
from jax.experimental.pallas import tpu as pltpu
from jax.experimental.pallas import tpu_sc as plsc

B, N, F, H = 256, 128, 9, 768
R = N + 1

_info = plsc.get_sparse_core_info()
NC, NS = _info.num_cores, _info.num_subcores
NW = NC * NS
T = 8
KPG = N // T
GPW = B // NW
CHUNKS = GPW * KPG
NFULL = B * KPG
CW = T * F + 2 * T
LANES = 16
JCH = H // LANES


def _sc_kernel(comb_hbm, atom_hbm, inw_hbm, outw_hbm, gtok_hbm, out_hbm,
               idx_a, idx_b, rows_a, rows_b, deg_v, acc_v, gt_v,
               sem_a, sem_b, sem_d, sem_o, sem_ia, sem_ib):
    wid = lax.axis_index("s") * NC + lax.axis_index("c")
    gc0 = wid * CHUNKS
    g0 = wid * GPW

    def stage_row(gc, idx_ref, sem):
        pltpu.async_copy(comb_hbm.at[pl.ds(gc * CW, CW)], idx_ref, sem)

    def wait_idx(idx_ref, sem):
        pltpu.make_async_copy(comb_hbm.at[pl.ds(0, CW)], idx_ref,
                              sem).wait()

    def fire_atoms(idx_ref, rows_ref, sem):
        pltpu.async_copy(atom_hbm.at[idx_ref.at[pl.ds(0, T * F)]],
                         rows_ref, sem)

    def wait_atoms(rows_ref, sem):
        pltpu.make_async_copy(atom_hbm.at[pl.ds(0, T * F)], rows_ref,
                              sem).wait()

    def fire_deg(idx_ref):
        pltpu.async_copy(inw_hbm.at[idx_ref.at[pl.ds(T * F, T)]],
                         deg_v.at[pl.ds(0, T)], sem_d)
        pltpu.async_copy(outw_hbm.at[idx_ref.at[pl.ds(T * F + T, T)]],
                         deg_v.at[pl.ds(T, T)], sem_d)

    def wait_deg():
        pltpu.make_async_copy(inw_hbm.at[pl.ds(0, T)],
                              deg_v.at[pl.ds(0, T)], sem_d).wait()
        pltpu.make_async_copy(outw_hbm.at[pl.ds(0, T)],
                              deg_v.at[pl.ds(T, T)], sem_d).wait()

    def wait_out():
        pltpu.make_async_copy(out_hbm.at[0, pl.ds(0, T)], acc_v,
                              sem_o).wait()

    def compute(rows_ref):
        def lane_body(j, _):
            s = pl.ds(j * LANES, LANES)
            accs = [rows_ref[t * F, s] for t in range(T)]
            for f in range(1, F):
                for t in range(T):
                    accs[t] = accs[t] + rows_ref[t * F + f, s]
            for t in range(T):
                accs[t] = accs[t] + deg_v[t, s]
            for t in range(T):
                accs[t] = accs[t] + deg_v[t + T, s]
            for t in range(T):
                acc_v[t, s] = accs[t]
            return 0

        lax.fori_loop(0, JCH, lane_body, 0)

    def store_out(c):
        g = g0 + lax.shift_right_logical(c, 4)
        r0 = pl.multiple_of(lax.shift_left(lax.bitwise_and(c, 15), 3), T)
        pltpu.async_copy(acc_v, out_hbm.at[g, pl.ds(r0, T)], sem_o)

    stage_row(gc0, idx_a, sem_ia)
    wait_idx(idx_a, sem_ia)
    fire_atoms(idx_a, rows_a, sem_a)
    fire_deg(idx_a)
    stage_row(gc0 + 1, idx_b, sem_ib)

    pltpu.sync_copy(gtok_hbm, gt_v)

    def pair_body(i, _):
        c0 = i * 2
        wait_idx(idx_b, sem_ib)
        fire_atoms(idx_b, rows_b, sem_b)
        wait_atoms(rows_a, sem_a)
        wait_deg()

        @pl.when(i > 0)
        def _():
            wait_out()

        stage_row(gc0 + c0 + 2, idx_a, sem_ia)
        compute(rows_a)
        fire_deg(idx_b)
        store_out(c0)
        wait_idx(idx_a, sem_ia)
        fire_atoms(idx_a, rows_a, sem_a)
        wait_atoms(rows_b, sem_b)
        wait_deg()
        wait_out()
        stage_row(gc0 + c0 + 3, idx_b, sem_ib)
        compute(rows_b)
        fire_deg(idx_a)
        store_out(c0 + 1)
        return 0

    lax.fori_loop(0, CHUNKS // 2, pair_body, 0)

    wait_idx(idx_b, sem_ib)
    wait_atoms(rows_a, sem_a)
    wait_deg()
    wait_out()

    for gi in range(GPW):
        pltpu.sync_copy(gt_v, out_hbm.at[g0 + gi, pl.ds(0, 1)])

    stage_row(NFULL + wid, idx_a, sem_ia)
    wait_idx(idx_a, sem_ia)
    fire_atoms(idx_a, rows_a, sem_a)
    fire_deg(idx_a)
    wait_atoms(rows_a, sem_a)
    wait_deg()
    compute(rows_a)
    for gi in range(GPW):
        pltpu.sync_copy(acc_v.at[pl.ds(gi, 1)],
                        out_hbm.at[g0 + gi, pl.ds(N, 1)])


@jax.jit
def _run(comb, atom_w, in_w, out_w, gtok):
    mesh = plsc.VectorSubcoreMesh(core_axis_name="c", subcore_axis_name="s")
    kfn = functools.partial(
        pl.kernel,
        mesh=mesh,
        out_type=jax.ShapeDtypeStruct((B, R, H), jnp.float32),
        scratch_types=[
            pltpu.VMEM((CW,), jnp.int32),
            pltpu.VMEM((CW,), jnp.int32),
            pltpu.VMEM((T * F, H), jnp.float32),
            pltpu.VMEM((T * F, H), jnp.float32),
            pltpu.VMEM((2 * T, H), jnp.float32),
            pltpu.VMEM((T, H), jnp.float32),
            pltpu.VMEM((1, H), jnp.float32),
            pltpu.SemaphoreType.DMA,
            pltpu.SemaphoreType.DMA,
            pltpu.SemaphoreType.DMA,
            pltpu.SemaphoreType.DMA,
            pltpu.SemaphoreType.DMA,
            pltpu.SemaphoreType.DMA,
        ],
    )(_sc_kernel)
    return kfn(comb, atom_w, in_w, out_w, gtok)


def kernel(input_nodes, in_degree, out_degree, atom_w, in_w, out_w,
           graph_token_w):
    nodes = jnp.asarray(input_nodes, jnp.int32)
    ind = jnp.asarray(in_degree, jnp.int32)
    outd = jnp.asarray(out_degree, jnp.int32)
    atomsP = jnp.concatenate([jnp.zeros((B, 1, F), jnp.int32), nodes],
                             axis=1)
    insP = jnp.concatenate([jnp.zeros((B, 1), jnp.int32), ind], axis=1)
    outsP = jnp.concatenate([jnp.zeros((B, 1), jnp.int32), outd], axis=1)
    comb_full = jnp.concatenate(
        [atomsP[:, :N].reshape(B, KPG, T * F),
         insP[:, :N].reshape(B, KPG, T),
         outsP[:, :N].reshape(B, KPG, T)], axis=-1).reshape(NFULL, CW)
    comb_tail = jnp.concatenate(
        [atomsP[:, N].reshape(NW, GPW * F),
         insP[:, N].reshape(NW, GPW),
         outsP[:, N].reshape(NW, GPW)], axis=-1)
    comb = jnp.concatenate([comb_full, comb_tail], axis=0).reshape(-1)
    return _run(comb, atom_w, in_w, out_w, graph_token_w)
